# baseline (speedup 1.0000x reference)
import functools

import jax
import jax.numpy as jnp
from jax import lax
from jax.experimental import pallas as pl
from jax.experimental.pallas import tpu as pltpu
from jax.experimental.pallas import tpu_sc as plsc

N_NODES = 10000
DIM = 128
NUM_PATHS = 8
PATH_LEN = 4
NUM_SLOTS = NUM_PATHS * PATH_LEN
ALPHA = 0.5

NC, NS, LANES = 2, 16, 16
NW = NC * NS
CHUNK = 8
NODES_PER_W = 320
N_PAD = NODES_PER_W * NW
N_CHUNKS = NODES_PER_W // CHUNK
ROWS_PER_CHUNK = CHUNK * NUM_SLOTS
GATHER_SPLIT = 128
N_SUB = DIM // LANES


def _worker_id():
    return lax.axis_index("s") * NC + lax.axis_index("c")


def _sc_layer_body(feats_hbm, idx_hbm, w_hbm, sel_hbm, out_hbm,
                   idx_all, rows0, rows1, out0, out1, w_v, sel_v,
                   sem0, sem1, osem0, osem1):
    wid = _worker_id()
    pltpu.sync_copy(w_hbm, w_v)
    pltpu.sync_copy(sel_hbm, sel_v)
    idx_row0 = wid * (NODES_PER_W * NUM_SLOTS // GATHER_SPLIT)
    pltpu.sync_copy(
        idx_hbm.at[pl.ds(idx_row0, NODES_PER_W * NUM_SLOTS // GATHER_SPLIT)],
        idx_all)

    def gather(cc, rows_b, sem_b):
        pltpu.async_copy(feats_hbm.at[idx_all.at[2 * cc]],
                         rows_b.at[pl.ds(0, GATHER_SPLIT)], sem_b)
        pltpu.async_copy(feats_hbm.at[idx_all.at[2 * cc + 1]],
                         rows_b.at[pl.ds(GATHER_SPLIT, GATHER_SPLIT)], sem_b)

    def gather_wait(cc, rows_b, sem_b):
        pltpu.make_async_copy(feats_hbm.at[idx_all.at[2 * cc]],
                              rows_b.at[pl.ds(0, GATHER_SPLIT)], sem_b).wait()
        pltpu.make_async_copy(feats_hbm.at[idx_all.at[2 * cc + 1]],
                              rows_b.at[pl.ds(GATHER_SPLIT, GATHER_SPLIT)],
                              sem_b).wait()

    node_base = wid * NODES_PER_W
    gather(0, rows0, sem0)
    gather(1, rows1, sem1)

    @pl.loop(0, N_CHUNKS, step=2)
    def chunk_loop(c):
        for b, rows_v, sem_b, out_v, osem_b in (
                (0, rows0, sem0, out0, osem0), (1, rows1, sem1, out1, osem1)):
            cc = c + b
            node0 = node_base + cc * CHUNK
            gather_wait(cc, rows_v, sem_b)

            @pl.when(cc >= 2)
            def _():
                pltpu.make_async_copy(
                    out_v, out_hbm.at[pl.ds(node0, CHUNK)], osem_b).wait()

            for sub in range(N_SUB):
                col = sub * LANES
                ws = [[w_v[p, l, pl.ds(col, LANES)] for l in range(PATH_LEN)]
                      for p in range(NUM_PATHS)]
                sels = [sel_v[p, :] for p in range(NUM_PATHS)]

                @pl.loop(0, CHUNK)
                def node_loop(n, _col=col, _ws=ws, _sels=sels, _rows=rows_v,
                              _out=out_v):
                    base = n * NUM_SLOTS
                    acc0 = jnp.zeros((LANES,), jnp.float32)
                    acc1 = jnp.zeros((LANES,), jnp.float32)
                    for p in range(NUM_PATHS):
                        r = base + p * PATH_LEN
                        contrib = _ws[p][0] * _rows[r, pl.ds(_col, LANES)]
                        for l in range(1, PATH_LEN):
                            contrib = contrib + _ws[p][l] * _rows[r + l, pl.ds(_col, LANES)]
                        t = _sels[p] * contrib
                        acc0 = acc0 + t
                        acc1 = acc1 + (contrib - t)
                    _out[n, pl.ds(_col, LANES)] = acc0
                    _out[n, pl.ds(DIM + _col, LANES)] = acc1

            pltpu.async_copy(out_v, out_hbm.at[pl.ds(node0, CHUNK)], osem_b)

            @pl.when(cc + 2 < N_CHUNKS)
            def _():
                gather(cc + 2, rows_v, sem_b)

    pltpu.make_async_copy(
        out0, out_hbm.at[pl.ds(node_base, CHUNK)], osem0).wait()
    pltpu.make_async_copy(
        out1, out_hbm.at[pl.ds(node_base, CHUNK)], osem1).wait()


@jax.jit
def _sc_layer(feats_pad, idx2d, w, sel):
    mesh = plsc.VectorSubcoreMesh(
        core_axis_name="c", subcore_axis_name="s", num_cores=NC,
        num_subcores=NS)
    f = pl.kernel(
        _sc_layer_body,
        out_type=jax.ShapeDtypeStruct((N_PAD, 2 * DIM), jnp.float32),
        mesh=mesh,
        scratch_types=[
            pltpu.VMEM((NODES_PER_W * NUM_SLOTS // GATHER_SPLIT,
                        GATHER_SPLIT), jnp.int32),
            pltpu.VMEM((ROWS_PER_CHUNK, DIM), jnp.float32),
            pltpu.VMEM((ROWS_PER_CHUNK, DIM), jnp.float32),
            pltpu.VMEM((CHUNK, 2 * DIM), jnp.float32),
            pltpu.VMEM((CHUNK, 2 * DIM), jnp.float32),
            pltpu.VMEM((NUM_PATHS, PATH_LEN, DIM), jnp.float32),
            pltpu.VMEM((NUM_PATHS, LANES), jnp.float32),
            pltpu.SemaphoreType.DMA,
            pltpu.SemaphoreType.DMA,
            pltpu.SemaphoreType.DMA,
            pltpu.SemaphoreType.DMA,
        ],
    )
    return f(feats_pad, idx2d, w, sel)


def _mm_kernel(x_ref, w_ref, b_ref, o_ref):
    acc = jnp.dot(x_ref[...], w_ref[...], preferred_element_type=jnp.float32)
    o_ref[...] = jnp.maximum(acc + b_ref[...], 0.0)


def _mm_relu(x, W, b, rows_blk=1024):
    n, k = x.shape
    m = W.shape[1]
    return pl.pallas_call(
        _mm_kernel,
        grid=(n // rows_blk,),
        in_specs=[
            pl.BlockSpec((rows_blk, k), lambda i: (i, 0)),
            pl.BlockSpec((k, m), lambda i: (0, 0)),
            pl.BlockSpec((1, m), lambda i: (0, 0)),
        ],
        out_specs=pl.BlockSpec((rows_blk, m), lambda i: (i, 0)),
        out_shape=jax.ShapeDtypeStruct((n, m), jnp.float32),
    )(x, W, b.reshape(1, m))


def _fc_kernel(res_ref, w_ref, in_ref, o_ref):
    acc = jnp.dot(res_ref[...], w_ref[...], preferred_element_type=jnp.float32)
    o_ref[...] = ALPHA * in_ref[...] + (1.0 - ALPHA) * jnp.maximum(acc, 0.0)


def _fc_residual(res, Wfc, in_feats, rows_blk=1024):
    n, k = res.shape
    m = Wfc.shape[1]
    return pl.pallas_call(
        _fc_kernel,
        grid=(n // rows_blk,),
        in_specs=[
            pl.BlockSpec((rows_blk, k), lambda i: (i, 0)),
            pl.BlockSpec((k, m), lambda i: (0, 0)),
            pl.BlockSpec((rows_blk, m), lambda i: (i, 0)),
        ],
        out_specs=pl.BlockSpec((rows_blk, m), lambda i: (i, 0)),
        out_shape=jax.ShapeDtypeStruct((n, m), jnp.float32),
    )(res, Wfc, in_feats)


def kernel(input_x, paths, path_types, W_in, b_in, fc_W, pw, W_out, b_out):
    num_layers = fc_W.shape[0]
    num_edge_types = pw.shape[1]

    ptypes = path_types.astype(jnp.int32)
    cnt = jnp.maximum(
        jnp.sum(ptypes[None, :] == jnp.arange(num_edge_types)[:, None],
                axis=1).astype(jnp.float32), 1.0)
    w_all = pw[:, ptypes, :, :] / cnt[ptypes][None, :, None, None]
    sel0 = (ptypes == 0).astype(jnp.float32)
    sel_b = jnp.broadcast_to(sel0[:, None], (NUM_PATHS, LANES))

    idx = jnp.transpose(paths.astype(jnp.int32), (1, 0, 2)).reshape(
        N_NODES, NUM_SLOTS)
    idx = jnp.pad(idx, ((0, N_PAD - N_NODES), (0, 0)))
    idx2d = idx.reshape(N_PAD * NUM_SLOTS // GATHER_SPLIT, GATHER_SPLIT)

    x_pad = jnp.pad(input_x, ((0, N_PAD - N_NODES), (0, 0)))
    in_feats = _mm_relu(x_pad, W_in, b_in)

    feats = in_feats
    for i in range(num_layers):
        res = _sc_layer(feats, idx2d, w_all[i], sel_b)
        feats = _fc_residual(res, fc_W[i], in_feats)

    out = _mm_relu(feats, W_out, b_out)
    return out[:N_NODES]


# speedup vs baseline: 3.1312x; 3.1312x over previous
import functools

import jax
import jax.numpy as jnp
from jax import lax
from jax.experimental import pallas as pl
from jax.experimental.pallas import tpu as pltpu
from jax.experimental.pallas import tpu_sc as plsc

N_NODES = 10000
DIM = 128
NUM_PATHS = 8
PATH_LEN = 4
NUM_SLOTS = NUM_PATHS * PATH_LEN
ALPHA = 0.5

NC, NS, LANES = 2, 16, 16
NW = NC * NS
CHUNK = 2
NODES_PER_W = 320
N_PAD = NODES_PER_W * NW
N_CHUNKS = NODES_PER_W // CHUNK
ROWS_PER_CHUNK = CHUNK * NUM_SLOTS
NBUF = 2
N_SUB = DIM // LANES


def _worker_id():
    return lax.axis_index("s") * NC + lax.axis_index("c")


def _sc_layer_body(feats_hbm, idx_hbm, w_hbm, sel_hbm, out_hbm,
                   idx_all, tbl_sp, w_v, sel_v, *bufs):
    rows = bufs[0:NBUF]
    outs = bufs[NBUF:2 * NBUF]
    gsems = bufs[2 * NBUF:3 * NBUF]
    osems = bufs[3 * NBUF:4 * NBUF]

    wid = _worker_id()
    pltpu.sync_copy(w_hbm, w_v)
    pltpu.sync_copy(sel_hbm, sel_v)
    idx_row0 = wid * N_CHUNKS
    pltpu.sync_copy(idx_hbm.at[pl.ds(idx_row0, N_CHUNKS)], idx_all)

    sid = lax.axis_index("s")
    stripe = N_PAD // NS
    pltpu.sync_copy(feats_hbm.at[pl.ds(sid * stripe, stripe)],
                    tbl_sp.at[pl.ds(sid * stripe, stripe)])
    plsc.subcore_barrier()

    def gather(cc, b):
        pltpu.async_copy(tbl_sp.at[idx_all.at[cc]], rows[b], gsems[b])

    def gather_wait(cc, b):
        pltpu.make_async_copy(
            tbl_sp.at[idx_all.at[cc]], rows[b], gsems[b]).wait()

    node_base = wid * NODES_PER_W
    for b in range(NBUF):
        gather(b, b)

    @pl.loop(0, N_CHUNKS, step=NBUF)
    def chunk_loop(c):
        for b in range(NBUF):
            cc = c + b
            node0 = node_base + cc * CHUNK
            gather_wait(cc, b)

            @pl.when(cc >= NBUF)
            def _(b=b):
                pltpu.make_async_copy(
                    outs[b], out_hbm.at[pl.ds(node0, CHUNK)], osems[b]).wait()

            for sub in range(N_SUB):
                col = sub * LANES
                ws = [[w_v[p, l, pl.ds(col, LANES)] for l in range(PATH_LEN)]
                      for p in range(NUM_PATHS)]
                sels = [sel_v[p, :] for p in range(NUM_PATHS)]

                @pl.loop(0, CHUNK)
                def node_loop(n, _col=col, _ws=ws, _sels=sels, _rows=rows[b],
                              _out=outs[b]):
                    base = n * NUM_SLOTS
                    acc0 = None
                    tot = None
                    for p in range(NUM_PATHS):
                        r = base + p * PATH_LEN
                        contrib = _ws[p][0] * _rows[r, pl.ds(_col, LANES)]
                        for l in range(1, PATH_LEN):
                            contrib = contrib + _ws[p][l] * _rows[
                                r + l, pl.ds(_col, LANES)]
                        t = _sels[p] * contrib
                        acc0 = t if acc0 is None else acc0 + t
                        tot = contrib if tot is None else tot + contrib
                    _out[n, pl.ds(_col, LANES)] = acc0
                    _out[n, pl.ds(DIM + _col, LANES)] = tot - acc0

            pltpu.async_copy(outs[b], out_hbm.at[pl.ds(node0, CHUNK)],
                             osems[b])

            @pl.when(cc + NBUF < N_CHUNKS)
            def _(cc=cc, b=b):
                gather(cc + NBUF, b)

    for b in range(NBUF):
        pltpu.make_async_copy(
            outs[b], out_hbm.at[pl.ds(node_base, CHUNK)], osems[b]).wait()


@jax.jit
def _sc_layer(feats_pad, idx2d, w, sel):
    mesh = plsc.VectorSubcoreMesh(
        core_axis_name="c", subcore_axis_name="s", num_cores=NC,
        num_subcores=NS)
    scratch = [
        pltpu.VMEM((N_CHUNKS, ROWS_PER_CHUNK), jnp.int32),
        pltpu.VMEM_SHARED((N_PAD, DIM), jnp.float32),
        pltpu.VMEM((NUM_PATHS, PATH_LEN, DIM), jnp.float32),
        pltpu.VMEM((NUM_PATHS, LANES), jnp.float32),
    ]
    scratch += [pltpu.VMEM((ROWS_PER_CHUNK, DIM), jnp.float32)
                for _ in range(NBUF)]
    scratch += [pltpu.VMEM((CHUNK, 2 * DIM), jnp.float32)
                for _ in range(NBUF)]
    scratch += [pltpu.SemaphoreType.DMA for _ in range(2 * NBUF)]
    f = pl.kernel(
        _sc_layer_body,
        out_type=jax.ShapeDtypeStruct((N_PAD, 2 * DIM), jnp.float32),
        mesh=mesh,
        scratch_types=scratch,
    )
    return f(feats_pad, idx2d, w, sel)


def _mm_kernel(x_ref, w_ref, b_ref, o_ref):
    acc = jnp.dot(x_ref[...], w_ref[...], preferred_element_type=jnp.float32)
    o_ref[...] = jnp.maximum(acc + b_ref[...], 0.0)


def _mm_relu(x, W, b, rows_blk=1024):
    n, k = x.shape
    m = W.shape[1]
    return pl.pallas_call(
        _mm_kernel,
        grid=(n // rows_blk,),
        in_specs=[
            pl.BlockSpec((rows_blk, k), lambda i: (i, 0)),
            pl.BlockSpec((k, m), lambda i: (0, 0)),
            pl.BlockSpec((1, m), lambda i: (0, 0)),
        ],
        out_specs=pl.BlockSpec((rows_blk, m), lambda i: (i, 0)),
        out_shape=jax.ShapeDtypeStruct((n, m), jnp.float32),
    )(x, W, b.reshape(1, m))


def _fc_kernel(res_ref, w_ref, in_ref, o_ref):
    acc = jnp.dot(res_ref[...], w_ref[...], preferred_element_type=jnp.float32)
    o_ref[...] = ALPHA * in_ref[...] + (1.0 - ALPHA) * jnp.maximum(acc, 0.0)


def _fc_residual(res, Wfc, in_feats, rows_blk=1024):
    n, k = res.shape
    m = Wfc.shape[1]
    return pl.pallas_call(
        _fc_kernel,
        grid=(n // rows_blk,),
        in_specs=[
            pl.BlockSpec((rows_blk, k), lambda i: (i, 0)),
            pl.BlockSpec((k, m), lambda i: (0, 0)),
            pl.BlockSpec((rows_blk, m), lambda i: (i, 0)),
        ],
        out_specs=pl.BlockSpec((rows_blk, m), lambda i: (i, 0)),
        out_shape=jax.ShapeDtypeStruct((n, m), jnp.float32),
    )(res, Wfc, in_feats)


def kernel(input_x, paths, path_types, W_in, b_in, fc_W, pw, W_out, b_out):
    num_layers = fc_W.shape[0]
    num_edge_types = pw.shape[1]

    ptypes = path_types.astype(jnp.int32)
    cnt = jnp.maximum(
        jnp.sum(ptypes[None, :] == jnp.arange(num_edge_types)[:, None],
                axis=1).astype(jnp.float32), 1.0)
    w_all = pw[:, ptypes, :, :] / cnt[ptypes][None, :, None, None]
    sel0 = (ptypes == 0).astype(jnp.float32)
    sel_b = jnp.broadcast_to(sel0[:, None], (NUM_PATHS, LANES))

    idx = jnp.transpose(paths.astype(jnp.int32), (1, 0, 2)).reshape(
        N_NODES, NUM_SLOTS)
    idx = jnp.pad(idx, ((0, N_PAD - N_NODES), (0, 0)))
    idx2d = idx.reshape(N_PAD * NUM_SLOTS // ROWS_PER_CHUNK, ROWS_PER_CHUNK)

    x_pad = jnp.pad(input_x, ((0, N_PAD - N_NODES), (0, 0)))
    in_feats = _mm_relu(x_pad, W_in, b_in)

    feats = in_feats
    for i in range(num_layers):
        res = _sc_layer(feats, idx2d, w_all[i], sel_b)
        feats = _fc_residual(res, fc_W[i], in_feats)

    out = _mm_relu(feats, W_out, b_out)
    return out[:N_NODES]


# speedup vs baseline: 3.2174x; 1.0275x over previous
import functools

import jax
import jax.numpy as jnp
from jax import lax
from jax.experimental import pallas as pl
from jax.experimental.pallas import tpu as pltpu
from jax.experimental.pallas import tpu_sc as plsc

N_NODES = 10000
DIM = 128
NUM_PATHS = 8
PATH_LEN = 4
NUM_SLOTS = NUM_PATHS * PATH_LEN
ALPHA = 0.5

NC, NS, LANES = 2, 16, 16
NW = NC * NS
CHUNK = 2
NODES_PER_W = 320
N_PAD = NODES_PER_W * NW
N_CHUNKS = NODES_PER_W // CHUNK
ROWS_PER_CHUNK = CHUNK * NUM_SLOTS
NBUF = 2
N_SUB = DIM // LANES


def _worker_id():
    return lax.axis_index("s") * NC + lax.axis_index("c")


def _sc_layer_body(feats_hbm, idx_hbm, w_hbm, sel_hbm, out_hbm,
                   idx_all, tbl_sp, w_v, sel_v, *bufs):
    rows = bufs[0:NBUF]
    outs = bufs[NBUF:2 * NBUF]
    gsems = bufs[2 * NBUF:3 * NBUF]
    osems = bufs[3 * NBUF:4 * NBUF]

    wid = _worker_id()
    pltpu.sync_copy(w_hbm, w_v)
    pltpu.sync_copy(sel_hbm, sel_v)
    idx_row0 = wid * N_CHUNKS
    pltpu.sync_copy(idx_hbm.at[pl.ds(idx_row0, N_CHUNKS)], idx_all)

    sid = lax.axis_index("s")
    stripe = N_PAD // NS
    pltpu.sync_copy(feats_hbm.at[pl.ds(sid * stripe, stripe)],
                    tbl_sp.at[pl.ds(sid * stripe, stripe)])
    plsc.subcore_barrier()

    def gather(cc, b):
        pltpu.async_copy(tbl_sp.at[idx_all.at[cc]], rows[b], gsems[b])

    def gather_wait(cc, b):
        pltpu.make_async_copy(
            tbl_sp.at[idx_all.at[cc]], rows[b], gsems[b]).wait()

    node_base = wid * NODES_PER_W
    for b in range(NBUF):
        gather(b, b)

    @pl.loop(0, N_CHUNKS, step=NBUF)
    def chunk_loop(c):
        for b in range(NBUF):
            cc = c + b
            node0 = node_base + cc * CHUNK
            gather_wait(cc, b)

            @pl.when(cc >= NBUF)
            def _(b=b):
                pltpu.make_async_copy(
                    outs[b], out_hbm.at[pl.ds(node0, CHUNK)], osems[b]).wait()

            for sub in range(N_SUB):
                col = sub * LANES
                ws = [[w_v[p, l, pl.ds(col, LANES)] for l in range(PATH_LEN)]
                      for p in range(NUM_PATHS)]
                sels = [sel_v[p, :] for p in range(NUM_PATHS)]

                @pl.loop(0, CHUNK)
                def node_loop(n, _col=col, _ws=ws, _sels=sels, _rows=rows[b],
                              _out=outs[b]):
                    base = n * NUM_SLOTS
                    acc0 = None
                    tot = None
                    for p in range(NUM_PATHS):
                        r = base + p * PATH_LEN
                        contrib = _ws[p][0] * _rows[r, pl.ds(_col, LANES)]
                        for l in range(1, PATH_LEN):
                            contrib = contrib + _ws[p][l] * _rows[
                                r + l, pl.ds(_col, LANES)]
                        t = _sels[p] * contrib
                        acc0 = t if acc0 is None else acc0 + t
                        tot = contrib if tot is None else tot + contrib
                    _out[n, pl.ds(_col, LANES)] = acc0
                    _out[n, pl.ds(DIM + _col, LANES)] = tot - acc0

            pltpu.async_copy(outs[b], out_hbm.at[pl.ds(node0, CHUNK)],
                             osems[b])

            @pl.when(cc + NBUF < N_CHUNKS)
            def _(cc=cc, b=b):
                gather(cc + NBUF, b)

    for b in range(NBUF):
        pltpu.make_async_copy(
            outs[b], out_hbm.at[pl.ds(node_base, CHUNK)], osems[b]).wait()


@jax.jit
def _sc_layer(feats_pad, idx2d, w, sel):
    mesh = plsc.VectorSubcoreMesh(
        core_axis_name="c", subcore_axis_name="s", num_cores=NC,
        num_subcores=NS)
    scratch = [
        pltpu.VMEM((N_CHUNKS, ROWS_PER_CHUNK), jnp.int32),
        pltpu.VMEM_SHARED((N_PAD, DIM), jnp.float32),
        pltpu.VMEM((NUM_PATHS, PATH_LEN, DIM), jnp.float32),
        pltpu.VMEM((NUM_PATHS, LANES), jnp.float32),
    ]
    scratch += [pltpu.VMEM((ROWS_PER_CHUNK, DIM), jnp.float32)
                for _ in range(NBUF)]
    scratch += [pltpu.VMEM((CHUNK, 2 * DIM), jnp.float32)
                for _ in range(NBUF)]
    scratch += [pltpu.SemaphoreType.DMA for _ in range(2 * NBUF)]
    f = pl.kernel(
        _sc_layer_body,
        out_type=jax.ShapeDtypeStruct((N_PAD, 2 * DIM), jnp.float32),
        mesh=mesh,
        scratch_types=scratch,
    )
    return f(feats_pad, idx2d, w, sel)


def _mm_kernel(x_ref, w_ref, b_ref, o_ref):
    acc = jnp.dot(x_ref[...], w_ref[...], preferred_element_type=jnp.float32)
    o_ref[...] = jnp.maximum(acc + b_ref[...], 0.0)


def _mm_relu(x, W, b, rows_blk=1024):
    n, k = x.shape
    m = W.shape[1]
    return pl.pallas_call(
        _mm_kernel,
        grid=(n // rows_blk,),
        in_specs=[
            pl.BlockSpec((rows_blk, k), lambda i: (i, 0)),
            pl.BlockSpec((k, m), lambda i: (0, 0)),
            pl.BlockSpec((1, m), lambda i: (0, 0)),
        ],
        out_specs=pl.BlockSpec((rows_blk, m), lambda i: (i, 0)),
        out_shape=jax.ShapeDtypeStruct((n, m), jnp.float32),
    )(x, W, b.reshape(1, m))


def _fc_kernel(res_ref, w_ref, in_ref, o_ref):
    acc = jnp.dot(res_ref[...], w_ref[...], preferred_element_type=jnp.float32)
    o_ref[...] = ALPHA * in_ref[...] + (1.0 - ALPHA) * jnp.maximum(acc, 0.0)


def _fc_residual(res, Wfc, in_feats, rows_blk=1024):
    n, k = res.shape
    m = Wfc.shape[1]
    return pl.pallas_call(
        _fc_kernel,
        grid=(n // rows_blk,),
        in_specs=[
            pl.BlockSpec((rows_blk, k), lambda i: (i, 0)),
            pl.BlockSpec((k, m), lambda i: (0, 0)),
            pl.BlockSpec((rows_blk, m), lambda i: (i, 0)),
        ],
        out_specs=pl.BlockSpec((rows_blk, m), lambda i: (i, 0)),
        out_shape=jax.ShapeDtypeStruct((n, m), jnp.float32),
    )(res, Wfc, in_feats)


def _fc_out_kernel(res_ref, w_ref, in_ref, wo_ref, bo_ref, o_ref):
    acc = jnp.dot(res_ref[...], w_ref[...], preferred_element_type=jnp.float32)
    feats = ALPHA * in_ref[...] + (1.0 - ALPHA) * jnp.maximum(acc, 0.0)
    acc2 = jnp.dot(feats, wo_ref[...], preferred_element_type=jnp.float32)
    o_ref[...] = jnp.maximum(acc2 + bo_ref[...], 0.0)


def _fc_residual_out(res, Wfc, in_feats, W_out, b_out, rows_blk=1024):
    n, k = res.shape
    m = Wfc.shape[1]
    mo = W_out.shape[1]
    return pl.pallas_call(
        _fc_out_kernel,
        grid=(n // rows_blk,),
        in_specs=[
            pl.BlockSpec((rows_blk, k), lambda i: (i, 0)),
            pl.BlockSpec((k, m), lambda i: (0, 0)),
            pl.BlockSpec((rows_blk, m), lambda i: (i, 0)),
            pl.BlockSpec((m, mo), lambda i: (0, 0)),
            pl.BlockSpec((1, mo), lambda i: (0, 0)),
        ],
        out_specs=pl.BlockSpec((rows_blk, mo), lambda i: (i, 0)),
        out_shape=jax.ShapeDtypeStruct((n, mo), jnp.float32),
    )(res, Wfc, in_feats, W_out, b_out.reshape(1, mo))


def kernel(input_x, paths, path_types, W_in, b_in, fc_W, pw, W_out, b_out):
    num_layers = fc_W.shape[0]
    num_edge_types = pw.shape[1]

    ptypes = path_types.astype(jnp.int32)
    cnt = jnp.maximum(
        jnp.sum(ptypes[None, :] == jnp.arange(num_edge_types)[:, None],
                axis=1).astype(jnp.float32), 1.0)
    w_all = pw[:, ptypes, :, :] / cnt[ptypes][None, :, None, None]
    sel0 = (ptypes == 0).astype(jnp.float32)
    sel_b = jnp.broadcast_to(sel0[:, None], (NUM_PATHS, LANES))

    idx = jnp.transpose(paths.astype(jnp.int32), (1, 0, 2)).reshape(
        N_NODES, NUM_SLOTS)
    idx = jnp.pad(idx, ((0, N_PAD - N_NODES), (0, 0)))
    idx2d = idx.reshape(N_PAD * NUM_SLOTS // ROWS_PER_CHUNK, ROWS_PER_CHUNK)

    x_pad = jnp.pad(input_x, ((0, N_PAD - N_NODES), (0, 0)))
    in_feats = _mm_relu(x_pad, W_in, b_in)

    feats = in_feats
    for i in range(num_layers - 1):
        res = _sc_layer(feats, idx2d, w_all[i], sel_b)
        feats = _fc_residual(res, fc_W[i], in_feats)

    res = _sc_layer(feats, idx2d, w_all[num_layers - 1], sel_b)
    out = _fc_residual_out(res, fc_W[num_layers - 1], in_feats, W_out, b_out)
    return out[:N_NODES]
